# Initial kernel scaffold; baseline (speedup 1.0000x reference)
import jax
import jax.numpy as jnp
from jax.experimental import pallas as pl


def kernel(feats, segment_ids, num_segments, W, b):
    raise NotImplementedError("write your pallas kernel here")


# speedup vs baseline: 2.7332x; 2.7332x over previous
import functools

import jax
import jax.numpy as jnp
from jax import lax
from jax.experimental import pallas as pl
from jax.experimental.pallas import tpu as pltpu
from jax.experimental.pallas import tpu_sc as plsc

_N = 100000
_D = 128
_S = 1024
_CH = 128
_NCHUNKS = (_N + _CH - 1) // _CH
_NFULL = _N // _CH
_TAIL = _N - _NFULL * _CH
_NW = 32
_ZROWS = _S // 16

_BN = 1000


def _tc_weight_body(f_ref, w_ref, b_ref, o_ref):
    f = f_ref[...]
    x = lax.dot_general(f, w_ref[...], (((1,), (0,)), ((), ())),
                        preferred_element_type=jnp.float32)
    g = jax.nn.sigmoid(x + b_ref[...])
    o_ref[...] = f * g


def _tc_weight(feats, w, b2):
    n, d = feats.shape
    return pl.pallas_call(
        _tc_weight_body,
        grid=(n // _BN,),
        in_specs=[
            pl.BlockSpec((_BN, d), lambda i: (i, 0)),
            pl.BlockSpec((d, 1), lambda i: (0, 0)),
            pl.BlockSpec((1, 1), lambda i: (0, 0)),
        ],
        out_specs=pl.BlockSpec((_BN, d), lambda i: (i, 0)),
        out_shape=jax.ShapeDtypeStruct((n, d), jnp.float32),
    )(feats, w, b2)


def _combine_body(p_ref, o_ref):
    o_ref[...] = p_ref[0] + p_ref[1]


def _tc_combine(parts):
    return pl.pallas_call(
        _combine_body,
        out_shape=jax.ShapeDtypeStruct((_S, _D), jnp.float32),
    )(parts)


def _sc_scatter_build():
    mesh = plsc.VectorSubcoreMesh(core_axis_name="c", subcore_axis_name="s")

    @functools.partial(
        pl.kernel,
        mesh=mesh,
        out_type=jax.ShapeDtypeStruct((2 * _S, _D), jnp.float32),
        scratch_types=[
            pltpu.VMEM((_CH, _D), jnp.float32),
            pltpu.VMEM((_CH,), jnp.int32),
            pltpu.VMEM((_TAIL,), jnp.int32),
            pltpu.VMEM_SHARED((_S, _D), jnp.float32),
        ],
    )
    def _sc_scatter(wf_hbm, ids_hbm, zeros_hbm, out_hbm,
                    rows_v, idx_v, idxt_v, acc_sh):
        c = lax.axis_index("c")
        s = lax.axis_index("s")
        w = s * 2 + c

        pltpu.sync_copy(zeros_hbm, acc_sh.at[pl.ds(s * _ZROWS, _ZROWS)])
        plsc.subcore_barrier()

        nk = (_NCHUNKS - 1 - w) // _NW + 1

        def body(k, carry):
            cid = w + k * _NW
            base = cid * _CH

            @pl.when(cid < _NFULL)
            def _full():
                pltpu.sync_copy(ids_hbm.at[pl.ds(base, _CH)], idx_v)
                pltpu.sync_copy(wf_hbm.at[pl.ds(base, _CH)], rows_v)
                pltpu.sync_copy(rows_v, acc_sh.at[idx_v], add=True)

            @pl.when(cid == _NFULL)
            def _tail():
                pltpu.sync_copy(ids_hbm.at[pl.ds(base, _TAIL)], idxt_v)
                pltpu.sync_copy(wf_hbm.at[pl.ds(base, _TAIL)],
                                rows_v.at[pl.ds(0, _TAIL)])
                pltpu.sync_copy(rows_v.at[pl.ds(0, _TAIL)],
                                acc_sh.at[idxt_v], add=True)

            return carry

        lax.fori_loop(0, nk, body, 0)

        plsc.subcore_barrier()
        pltpu.sync_copy(acc_sh.at[pl.ds(s * _ZROWS, _ZROWS)],
                        out_hbm.at[pl.ds(c * _S + s * _ZROWS, _ZROWS)])

    return _sc_scatter


_sc_scatter_call = _sc_scatter_build()


def kernel(feats, segment_ids, num_segments, W, b):
    del num_segments
    wf = _tc_weight(feats, W, jnp.reshape(b, (1, 1)))
    ids = segment_ids.astype(jnp.int32)
    zeros = jnp.zeros((_ZROWS, _D), jnp.float32)
    parts = _sc_scatter_call(wf, ids, zeros)
    return _tc_combine(jnp.reshape(parts, (2, _S, _D)))
